# baseline (speedup 1.0000x reference)
import math

import jax
import jax.numpy as jnp
from jax import lax
from jax.experimental import pallas as pl
from jax.experimental.pallas import tpu as pltpu
from jax.experimental.pallas import tpu_sc as plsc

H = 16
NB = 32
MAXD = 128
K = 2048
Q = 2048
W = 4480
WL = 3072
NP = 4

NSHIFT = 8
GROUPS = 4
KPER = K // NSHIFT // GROUPS


JT = W // 128
ROWS = JT * H
SLABJ0 = 15
SLABR = 48


def _table_kernel(w_ref, u8_ref):
    s = pl.program_id(0)
    m = lax.broadcasted_iota(jnp.int32, (ROWS, 128), 0)
    cc = lax.broadcasted_iota(jnp.int32, (ROWS, 128), 1)
    i = 128 * (m // H) + cc
    u8_ref[...] = jnp.where(i < 2040 + s, w_ref[:, 31:32], w_ref[:, 15:16])

    m2 = lax.broadcasted_iota(jnp.int32, (SLABR, 128), 0)
    c2 = lax.broadcasted_iota(jnp.int32, (SLABR, 128), 1)
    i2 = 128 * (SLABJ0 + m2 // H) + c2
    n = i2 - (2040 + s)
    half = NB // 2
    ret = jnp.where(n < 0, half, 0)
    na = jnp.abs(n)
    max_exact = half // 2
    is_small = na < max_exact
    n_safe = jnp.maximum(na, 1)
    val = max_exact + (
        jnp.log(n_safe.astype(jnp.float32) / max_exact)
        / math.log(MAXD / max_exact)
        * (half - max_exact)
    ).astype(jnp.int32)
    val = jnp.minimum(val, half - 1)
    bucket = ret + jnp.where(is_small, na, val)
    acc = jnp.zeros((SLABR, 128), jnp.float32)
    r0 = SLABJ0 * H
    for b in range(NB):
        acc = acc + jnp.where(bucket == b, w_ref[r0 : r0 + SLABR, b : b + 1], 0.0)
    u8_ref[r0 : r0 + SLABR, :] = acc


KT = K // 8
QT = Q // 128
KT_PER = KT // 2


def _sc_body(u8_hbm, out_hbm, tbl, sem_in, sem_out):
    c = lax.axis_index("c")
    s = lax.axis_index("s")
    wid = s * 2 + c
    h = wid // 2
    kt0 = (wid % 2) * KT_PER
    jtb = 8 - (wid % 2) * 8

    loads = [
        pltpu.async_copy(
            u8_hbm.at[:, jtb + p + jtx, h, :],
            tbl.at[p, :, pl.ds(128 * jtx, 128)],
            sem_in,
        )
        for p in range(NP)
        for jtx in range(WL // 128)
    ]
    for d in loads:
        d.wait()

    def kt_body(j, carry):
        kt = kt0 + j
        i0 = 1016 - 8 * j

        def q_body(q4, carry2):
            pltpu.async_copy(
                tbl.at[:, :, pl.ds(i0 + 512 * q4, 128)],
                out_hbm.at[h, kt, pl.ds(NP * q4, NP)],
                sem_out,
            )
            return carry2

        return lax.fori_loop(0, QT // NP, q_body, carry)

    lax.fori_loop(0, KT_PER, kt_body, 0)

    def drain(idx, carry):
        pltpu.make_async_copy(
            out_hbm.at[0, 0, pl.ds(0, NP)],
            tbl.at[:, :, pl.ds(0, 128)],
            sem_out,
        ).wait()
        return carry

    lax.fori_loop(0, KT_PER * (QT // NP), drain, 0)


def kernel(key_len, query_len, weight):
    del key_len, query_len
    u8_2d = pl.pallas_call(
        _table_kernel,
        grid=(NSHIFT,),
        in_specs=[pl.BlockSpec((ROWS, NB), lambda s: (0, 0))],
        out_specs=pl.BlockSpec((ROWS, 128), lambda s: (s, 0)),
        out_shape=jax.ShapeDtypeStruct((NSHIFT * ROWS, 128), jnp.float32),
    )(jnp.tile(weight, (JT, 1)))
    u8 = u8_2d.reshape(NSHIFT, JT, H, 128)

    mesh = plsc.VectorSubcoreMesh(core_axis_name="c", subcore_axis_name="s")
    out5 = pl.kernel(
        _sc_body,
        out_type=jax.ShapeDtypeStruct((H, KT, QT, 8, 128), jnp.float32),
        mesh=mesh,
        scratch_types=[
            pltpu.VMEM((NP, NSHIFT, WL), jnp.float32),
            pltpu.SemaphoreType.DMA,
            pltpu.SemaphoreType.DMA,
        ],
        compiler_params=pltpu.CompilerParams(use_tc_tiling_on_sc=False),
    )(u8)
    return jnp.transpose(out5, (0, 1, 3, 2, 4)).reshape(H, K, Q)


# speedup vs baseline: 1.0050x; 1.0050x over previous
import math

import jax
import jax.numpy as jnp
from jax import lax
from jax.experimental import pallas as pl
from jax.experimental.pallas import tpu as pltpu
from jax.experimental.pallas import tpu_sc as plsc

H = 16
NB = 32
MAXD = 128
K = 2048
Q = 2048
W = 4480
WL = 3072
NP = 4

NSHIFT = 8
GROUPS = 4
KPER = K // NSHIFT // GROUPS


SLAB0 = 1920
SLABW = 384


def _table_kernel(w_ref, u8_ref):
    r = pl.program_id(0)
    i_full = lax.broadcasted_iota(jnp.int32, (H, W), 1)
    u8_ref[0] = jnp.where(
        i_full < 2040 + r, w_ref[:, 31:32], w_ref[:, 15:16]
    )

    i = lax.broadcasted_iota(jnp.int32, (H, SLABW), 1) + SLAB0
    n = i - (2040 + r)
    half = NB // 2
    ret = jnp.where(n < 0, half, 0)
    na = jnp.abs(n)
    max_exact = half // 2
    is_small = na < max_exact
    n_safe = jnp.maximum(na, 1)
    val = max_exact + (
        jnp.log(n_safe.astype(jnp.float32) / max_exact)
        / math.log(MAXD / max_exact)
        * (half - max_exact)
    ).astype(jnp.int32)
    val = jnp.minimum(val, half - 1)
    bucket = ret + jnp.where(is_small, na, val)
    acc = jnp.zeros((H, SLABW), jnp.float32)
    for b in range(NB):
        acc = acc + jnp.where(bucket == b, w_ref[:, b : b + 1], 0.0)
    u8_ref[0, :, SLAB0 : SLAB0 + SLABW] = acc


KT = K // 8
QT = Q // 128
KT_PER = KT // 2


def _sc_body(u8_hbm, out_hbm, tbl, sem_in, sem_out):
    c = lax.axis_index("c")
    s = lax.axis_index("s")
    wid = s * 2 + c
    h = wid // 2
    kt0 = (wid % 2) * KT_PER
    wl = 1024 - 8 * kt0

    loads = [
        pltpu.async_copy(
            u8_hbm.at[:, h, pl.ds(wl + 128 * p, WL)], tbl.at[p], sem_in
        )
        for p in range(NP)
    ]
    for d in loads:
        d.wait()

    def kt_body(j, carry):
        kt = kt0 + j
        i0 = 1016 - 8 * j

        def q_body(q4, carry2):
            pltpu.async_copy(
                tbl.at[:, :, pl.ds(i0 + 512 * q4, 128)],
                out_hbm.at[h, kt, pl.ds(NP * q4, NP)],
                sem_out,
            )
            return carry2

        return lax.fori_loop(0, QT // NP, q_body, carry)

    lax.fori_loop(0, KT_PER, kt_body, 0)

    def drain(idx, carry):
        pltpu.make_async_copy(
            out_hbm.at[0, 0, pl.ds(0, NP)],
            tbl.at[:, :, pl.ds(0, 128)],
            sem_out,
        ).wait()
        return carry

    lax.fori_loop(0, KT_PER * (QT // NP), drain, 0)


def kernel(key_len, query_len, weight):
    del key_len, query_len
    u8 = pl.pallas_call(
        _table_kernel,
        grid=(NSHIFT,),
        in_specs=[pl.BlockSpec((H, NB), lambda r: (0, 0))],
        out_specs=pl.BlockSpec((1, H, W), lambda r: (r, 0, 0)),
        out_shape=jax.ShapeDtypeStruct((NSHIFT, H, W), jnp.float32),
    )(weight)

    mesh = plsc.VectorSubcoreMesh(core_axis_name="c", subcore_axis_name="s")
    out5 = pl.kernel(
        _sc_body,
        out_type=jax.ShapeDtypeStruct((H, KT, QT, 8, 128), jnp.float32),
        mesh=mesh,
        scratch_types=[
            pltpu.VMEM((NP, NSHIFT, WL), jnp.float32),
            pltpu.SemaphoreType.DMA,
            pltpu.SemaphoreType.DMA,
        ],
        compiler_params=pltpu.CompilerParams(use_tc_tiling_on_sc=False),
    )(u8)
    return jnp.transpose(out5, (0, 1, 3, 2, 4)).reshape(H, K, Q)


# speedup vs baseline: 1.0117x; 1.0067x over previous
import math

import jax
import jax.numpy as jnp
from jax import lax
from jax.experimental import pallas as pl
from jax.experimental.pallas import tpu as pltpu
from jax.experimental.pallas import tpu_sc as plsc

H = 16
NB = 32
MAXD = 128
K = 2048
Q = 2048
W = 4480
WL = 3072
NP = 4

NSHIFT = 8
SLAB0 = 1920
SLABW = 384


def _table_kernel(w_ref, u8_ref):
    r = pl.program_id(0)
    i_full = lax.broadcasted_iota(jnp.int32, (H, W), 1)
    u8_ref[0] = jnp.where(
        i_full < 2040 + r, w_ref[:, 31:32], w_ref[:, 15:16]
    )

    i = lax.broadcasted_iota(jnp.int32, (H, SLABW), 1) + SLAB0
    n = i - (2040 + r)
    half = NB // 2
    ret = jnp.where(n < 0, half, 0)
    na = jnp.abs(n)
    max_exact = half // 2
    is_small = na < max_exact
    n_safe = jnp.maximum(na, 1)
    val = max_exact + (
        jnp.log(n_safe.astype(jnp.float32) / max_exact)
        / math.log(MAXD / max_exact)
        * (half - max_exact)
    ).astype(jnp.int32)
    val = jnp.minimum(val, half - 1)
    bucket = ret + jnp.where(is_small, na, val)
    acc = jnp.zeros((H, SLABW), jnp.float32)
    for b in range(NB):
        acc = acc + jnp.where(bucket == b, w_ref[:, b : b + 1], 0.0)
    u8_ref[0, :, SLAB0 : SLAB0 + SLABW] = acc


KT = K // 8
QT = Q // 128
KT_PER = KT // 2


def _sc_body(u8_hbm, out_hbm, tbl, sem_in, sem_out):
    c = lax.axis_index("c")
    s = lax.axis_index("s")
    wid = s * 2 + c
    h = wid // 2
    kt0 = (wid % 2) * KT_PER
    wl = 1024 - 8 * kt0

    loads = [
        pltpu.async_copy(
            u8_hbm.at[:, h, pl.ds(wl + 128 * p, WL)], tbl.at[p], sem_in
        )
        for p in range(NP)
    ]
    for d in loads:
        d.wait()

    def kt_body(j, carry):
        kt = kt0 + j
        i0 = 1016 - 8 * j

        def q_body(q4, carry2):
            pltpu.async_copy(
                tbl.at[:, :, pl.ds(i0 + 512 * q4, 128)],
                out_hbm.at[h, kt, pl.ds(NP * q4, NP)],
                sem_out,
            )
            return carry2

        return lax.fori_loop(0, QT // NP, q_body, carry)

    lax.fori_loop(0, KT_PER, kt_body, 0)

    def drain(idx, carry):
        pltpu.make_async_copy(
            out_hbm.at[0, 0, pl.ds(0, NP)],
            tbl.at[:, :, pl.ds(0, 128)],
            sem_out,
        ).wait()
        return carry

    lax.fori_loop(0, KT_PER * (QT // NP), drain, 0)


def kernel(key_len, query_len, weight):
    del key_len, query_len
    u8 = pl.pallas_call(
        _table_kernel,
        grid=(NSHIFT,),
        in_specs=[pl.BlockSpec((H, NB), lambda r: (0, 0))],
        out_specs=pl.BlockSpec((1, H, W), lambda r: (r, 0, 0)),
        out_shape=jax.ShapeDtypeStruct((NSHIFT, H, W), jnp.float32),
    )(weight)

    mesh = plsc.VectorSubcoreMesh(core_axis_name="c", subcore_axis_name="s")
    out5 = pl.kernel(
        _sc_body,
        out_type=jax.ShapeDtypeStruct((H, KT, QT, 8, 128), jnp.float32),
        mesh=mesh,
        scratch_types=[
            pltpu.VMEM((NP, NSHIFT, WL), jnp.float32),
            pltpu.SemaphoreType.DMA,
            pltpu.SemaphoreType.DMA,
        ],
        compiler_params=pltpu.CompilerParams(use_tc_tiling_on_sc=False),
    )(u8)
    return jnp.transpose(out5, (0, 1, 3, 2, 4)).reshape(H, K, Q)
